# baseline (speedup 1.0000x reference)
import functools

import jax
import jax.numpy as jnp
from jax import lax
from jax.experimental import pallas as pl
from jax.experimental.pallas import tpu as pltpu
from jax.experimental.pallas import tpu_sc as plsc

_NUM_EMBS = 1000000
_EMSIZE = 64
_T, _B = 512, 1024
_N = _T * _B

_NC = 2
_NS = 16
_NW = _NC * _NS
_PER_W = _N // _NW
_ROWS_W = _T // _NW
_C = 512
_NCHUNK = _PER_W // _C
_NBUF = 2
_L = 16


def _body(x_hbm, emb_hbm, out_hbm, x_v, idx_v,
          rows0, rows1, gsem0, gsem1, osem0, osem1):
    wid = lax.axis_index("s") * _NC + lax.axis_index("c")
    base = wid * _PER_W
    t0 = wid * _ROWS_W
    rows = (rows0, rows1)
    gsem = (gsem0, gsem1)
    osem = (osem0, osem1)

    pltpu.sync_copy(x_hbm.at[pl.ds(base, _PER_W)], x_v)

    def vec(v, c):
        xv = x_v[pl.ds(v * _L, _L)]
        bad = (xv != xv) | (jnp.abs(xv) == jnp.inf)
        iv = jnp.clip(xv, 0.0, float(_NUM_EMBS - 2)).astype(jnp.int32)
        idx_v[pl.ds(v * _L, _L)] = jnp.where(bad, _NUM_EMBS - 1, iv)
        return c

    lax.fori_loop(0, _PER_W // _L, vec, 0, unroll=8)

    def gather(c, b):
        return pltpu.make_async_copy(
            emb_hbm.at[idx_v.at[pl.ds(c * _C, _C)]], rows[b], gsem[b])

    def out_copy(g, b):
        return pltpu.make_async_copy(
            rows[b], out_hbm.at[t0 + g, pl.ds(b * _C, _C)], osem[b])

    gather(0, 0).start()
    gather(1, 1).start()

    def group(g, carry):
        for b in range(_NBUF):
            c = g * _NBUF + b
            gather(c, b).wait()
            od = out_copy(g, b)
            od.start()
            od.wait()
            nxt = c + _NBUF

            @pl.when(nxt < _NCHUNK)
            def _():
                gather(nxt, b).start()
        return carry

    lax.fori_loop(0, _NCHUNK // _NBUF, group, 0)


@jax.jit
def _run(x_flat, embedding):
    mesh = plsc.VectorSubcoreMesh(core_axis_name="c", subcore_axis_name="s")
    return pl.kernel(
        _body,
        mesh=mesh,
        compiler_params=pltpu.CompilerParams(use_tc_tiling_on_sc=False),
        out_type=jax.ShapeDtypeStruct((_T, _B, _EMSIZE), jnp.float32),
        scratch_types=[
            pltpu.VMEM((_PER_W,), jnp.float32),
            pltpu.VMEM((_PER_W,), jnp.int32),
            pltpu.VMEM((_C, _EMSIZE), jnp.float32),
            pltpu.VMEM((_C, _EMSIZE), jnp.float32),
            pltpu.SemaphoreType.DMA,
            pltpu.SemaphoreType.DMA,
            pltpu.SemaphoreType.DMA,
            pltpu.SemaphoreType.DMA,
        ],
    )(x_flat, embedding)


def kernel(x, embedding, single_eval_pos):
    x_flat = lax.squeeze(x, dimensions=(2,)).reshape(_N)
    return _run(x_flat, embedding)


# speedup vs baseline: 1.2552x; 1.2552x over previous
import jax
import jax.numpy as jnp
from jax import lax
from jax.experimental import pallas as pl
from jax.experimental.pallas import tpu as pltpu
from jax.experimental.pallas import tpu_sc as plsc

_NUM_EMBS = 1000000
_EMSIZE = 64
_T, _B = 512, 1024
_N = _T * _B

_NC = 2
_NS = 16
_NW = _NC * _NS
_PER_W = _N // _NW
_ROWS_W = _T // _NW
_C = 512
_NCHUNK = _PER_W // _C
_NBUF = 2


def _body(idx_hbm, emb_hbm, out_hbm, idx_v,
          rows0, rows1, gsem0, gsem1, osem0, osem1):
    wid = lax.axis_index("s") * _NC + lax.axis_index("c")
    base = wid * _PER_W
    t0 = wid * _ROWS_W
    rows = (rows0, rows1)
    gsem = (gsem0, gsem1)
    osem = (osem0, osem1)

    pltpu.sync_copy(idx_hbm.at[pl.ds(base, _PER_W)], idx_v)

    def gather(c, b):
        return pltpu.make_async_copy(
            emb_hbm.at[idx_v.at[pl.ds(c * _C, _C)]], rows[b], gsem[b])

    def out_copy(g, b):
        return pltpu.make_async_copy(
            rows[b],
            out_hbm.at[t0 + g, pl.ds(b * _C, _C), pl.ds(0, _EMSIZE)],
            osem[b])

    gather(0, 0).start()
    gather(1, 1).start()

    def group(g, carry):
        for b in range(_NBUF):
            c = g * _NBUF + b
            gather(c, b).wait()
            od = out_copy(g, b)
            od.start()
            od.wait()
            nxt = c + _NBUF

            @pl.when(nxt < _NCHUNK)
            def _():
                gather(nxt, b).start()
        return carry

    lax.fori_loop(0, _NCHUNK // _NBUF, group, 0)


@jax.jit
def _run(idx_flat, embedding):
    mesh = plsc.VectorSubcoreMesh(core_axis_name="c", subcore_axis_name="s")
    return pl.kernel(
        _body,
        mesh=mesh,
        compiler_params=pltpu.CompilerParams(use_tc_tiling_on_sc=False),
        out_type=jax.ShapeDtypeStruct((_T, _B, 2 * _EMSIZE), jnp.float32),
        scratch_types=[
            pltpu.VMEM((_PER_W,), jnp.int32),
            pltpu.VMEM((_C, _EMSIZE), jnp.float32),
            pltpu.VMEM((_C, _EMSIZE), jnp.float32),
            pltpu.SemaphoreType.DMA,
            pltpu.SemaphoreType.DMA,
            pltpu.SemaphoreType.DMA,
            pltpu.SemaphoreType.DMA,
        ],
    )(idx_flat, embedding)


def kernel(x, embedding, single_eval_pos):
    xs = x[..., 0]
    bad = jnp.isnan(xs) | jnp.isinf(xs)
    idx = jnp.clip(xs, 0.0, float(_NUM_EMBS - 2)).astype(jnp.int32)
    idx = jnp.where(bad, _NUM_EMBS - 1, idx).reshape(_N)
    out128 = _run(idx, embedding)
    return out128[..., :_EMSIZE]
